# baseline (speedup 1.0000x reference)
import jax
import jax.numpy as jnp
from jax import lax
from jax.experimental import pallas as pl
from jax.experimental.pallas import tpu as pltpu
from jax.experimental.pallas import tpu_sc as plsc

B = 16384
F = 26
V = 100000
D = 16

NW = 32
BPW = B // NW
CHUNK = 128
KPW = BPW // CHUNK
NBLK = F * KPW
NB = 4
DEPTH = 3


def _body(x_hbm, tab_hbm, op_hbm, xv, fidx, gbuf, tbuf, gsem, osem):
    cid = lax.axis_index("c")
    sid = lax.axis_index("s")
    wid = sid * 2 + cid
    b0 = wid * BPW

    pltpu.sync_copy(x_hbm.at[pl.ds(b0 * F, BPW * F)], xv)

    iota = lax.iota(jnp.int32, 16)
    zeros = jnp.zeros((16,), jnp.int32)
    iota_f = iota * F

    def build_and_fire(blk):
        f = lax.shift_right_logical(blk, 2)
        k = lax.bitwise_and(blk, 3)
        slot = lax.bitwise_and(blk, NB - 1)
        for q in range(8):
            addrs = iota_f + ((k * CHUNK + q * 16) * F + f)
            fidx[slot, pl.ds(q * 16, 16)] = plsc.load_gather(xv, [addrs])
        pltpu.async_copy(tab_hbm.at[f].at[fidx.at[slot]], gbuf.at[slot], gsem)

    def drain_and_emit(blk):
        f = lax.shift_right_logical(blk, 2)
        k = lax.bitwise_and(blk, 3)
        slot = lax.bitwise_and(blk, NB - 1)
        bt = wid * KPW + k
        pltpu.make_async_copy(
            tab_hbm.at[0, pl.ds(0, CHUNK)], gbuf.at[slot], gsem
        ).wait()
        @pl.when(blk >= NB)
        def _():
            pltpu.make_async_copy(
                op_hbm.at[0, 0, 0], tbuf.at[slot, pl.ds(0, 8)], osem
            ).wait()
            pltpu.make_async_copy(
                op_hbm.at[0, 0, 0], tbuf.at[slot, pl.ds(8, 8)], osem
            ).wait()

        def tr(j, carry):
            v = gbuf[slot, j, :]
            plsc.store_scatter(tbuf.at[slot], [iota, zeros + j], v)
            return carry

        lax.fori_loop(0, CHUNK, tr, 0)
        pltpu.async_copy(tbuf.at[slot, pl.ds(0, 8)], op_hbm.at[f, 0, bt], osem)
        pltpu.async_copy(tbuf.at[slot, pl.ds(8, 8)], op_hbm.at[f, 1, bt], osem)

    for blk in range(DEPTH):
        build_and_fire(jnp.int32(blk))

    def loop_body(blk, carry):
        build_and_fire(blk)
        drain_and_emit(blk - DEPTH)
        return carry

    lax.fori_loop(DEPTH, NBLK, loop_body, 0)
    for r in range(DEPTH):
        drain_and_emit(jnp.int32(NBLK - DEPTH + r))
    for r in range(NB):
        pltpu.make_async_copy(
            op_hbm.at[0, 0, 0], tbuf.at[r, pl.ds(0, 8)], osem
        ).wait()
        pltpu.make_async_copy(
            op_hbm.at[0, 0, 0], tbuf.at[r, pl.ds(8, 8)], osem
        ).wait()


def kernel(X, tables):
    mesh = plsc.VectorSubcoreMesh(core_axis_name="c", subcore_axis_name="s")
    op = pl.kernel(
        _body,
        mesh=mesh,
        out_type=jax.ShapeDtypeStruct((F, 2, B // CHUNK, 8, CHUNK), jnp.float32),
        scratch_types=[
            pltpu.VMEM((BPW * F,), jnp.int32),
            pltpu.VMEM((NB, CHUNK), jnp.int32),
            pltpu.VMEM((NB, CHUNK, D), jnp.float32),
            pltpu.VMEM((NB, D, CHUNK), jnp.float32),
            pltpu.SemaphoreType.DMA,
            pltpu.SemaphoreType.DMA,
        ],
        compiler_params=pltpu.CompilerParams(
            use_tc_tiling_on_sc=False, needs_layout_passes=False
        ),
    )(X.astype(jnp.int32).reshape(B * F), tables)
    return (
        op.transpose(0, 1, 3, 2, 4)
        .reshape(F, D, B)
        .transpose(2, 0, 1)
    )


# speedup vs baseline: 1.0085x; 1.0085x over previous
import jax
import jax.numpy as jnp
from jax import lax
from jax.experimental import pallas as pl
from jax.experimental.pallas import tpu as pltpu
from jax.experimental.pallas import tpu_sc as plsc

B = 16384
F = 26
V = 100000
D = 16

NW = 32
BPW = B // NW
CHUNK = 128
KPW = BPW // CHUNK
NBLK = F * KPW
NB = 4
DEPTH = 3


def _body(x_hbm, tab_hbm, op_hbm, xv, fidx, ridx, gbuf, tbuf, gsem, osem):
    cid = lax.axis_index("c")
    sid = lax.axis_index("s")
    wid = sid * 2 + cid
    b0 = wid * BPW

    pltpu.sync_copy(x_hbm.at[pl.ds(b0 * F, BPW * F)], xv)

    iota = lax.iota(jnp.int32, 16)
    zeros = jnp.zeros((16,), jnp.int32)
    iota_f = iota * F

    def build_and_fire(blk):
        f = lax.shift_right_logical(blk, 2)
        k = lax.bitwise_and(blk, 3)
        slot = lax.bitwise_and(blk, NB - 1)
        fv = f * V
        for q in range(8):
            addrs = iota_f + ((k * CHUNK + q * 16) * F + f)
            g = plsc.load_gather(xv, [addrs]) + fv
            fidx[slot, pl.ds(q * 16, 16)] = g
            ridx[slot, pl.ds(q * 16, 16)] = lax.shift_right_logical(g, 3)
        pltpu.async_copy(tab_hbm.at[ridx.at[slot]], gbuf.at[slot], gsem)

    def drain_and_emit(blk):
        f = lax.shift_right_logical(blk, 2)
        k = lax.bitwise_and(blk, 3)
        slot = lax.bitwise_and(blk, NB - 1)
        bt = wid * KPW + k
        pltpu.make_async_copy(
            tab_hbm.at[pl.ds(0, CHUNK)], gbuf.at[slot], gsem
        ).wait()
        @pl.when(blk >= NB)
        def _():
            pltpu.make_async_copy(
                op_hbm.at[0, 0, 0], tbuf.at[slot, pl.ds(0, 8)], osem
            ).wait()
            pltpu.make_async_copy(
                op_hbm.at[0, 0, 0], tbuf.at[slot, pl.ds(8, 8)], osem
            ).wait()

        def tr(q, carry):
            off = lax.bitwise_and(fidx[slot, pl.ds(q * 16, 16)], 7) * D
            rows = iota + q * 16
            for d in range(D):
                tbuf[slot, d, pl.ds(q * 16, 16)] = plsc.load_gather(
                    gbuf.at[slot], [rows, off + d]
                )
            return carry

        lax.fori_loop(0, 8, tr, 0)
        pltpu.async_copy(tbuf.at[slot, pl.ds(0, 8)], op_hbm.at[f, 0, bt], osem)
        pltpu.async_copy(tbuf.at[slot, pl.ds(8, 8)], op_hbm.at[f, 1, bt], osem)

    for blk in range(DEPTH):
        build_and_fire(jnp.int32(blk))

    def loop_body(blk, carry):
        build_and_fire(blk)
        drain_and_emit(blk - DEPTH)
        return carry

    lax.fori_loop(DEPTH, NBLK, loop_body, 0)
    for r in range(DEPTH):
        drain_and_emit(jnp.int32(NBLK - DEPTH + r))
    for r in range(NB):
        pltpu.make_async_copy(
            op_hbm.at[0, 0, 0], tbuf.at[r, pl.ds(0, 8)], osem
        ).wait()
        pltpu.make_async_copy(
            op_hbm.at[0, 0, 0], tbuf.at[r, pl.ds(8, 8)], osem
        ).wait()


def kernel(X, tables):
    mesh = plsc.VectorSubcoreMesh(core_axis_name="c", subcore_axis_name="s")
    op = pl.kernel(
        _body,
        mesh=mesh,
        out_type=jax.ShapeDtypeStruct((F, 2, B // CHUNK, 8, CHUNK), jnp.float32),
        scratch_types=[
            pltpu.VMEM((BPW * F,), jnp.int32),
            pltpu.VMEM((NB, CHUNK), jnp.int32),
            pltpu.VMEM((NB, CHUNK), jnp.int32),
            pltpu.VMEM((NB, CHUNK, CHUNK), jnp.float32),
            pltpu.VMEM((NB, D, CHUNK), jnp.float32),
            pltpu.SemaphoreType.DMA,
            pltpu.SemaphoreType.DMA,
        ],
        compiler_params=pltpu.CompilerParams(
            use_tc_tiling_on_sc=False, needs_layout_passes=False
        ),
    )(X.astype(jnp.int32).reshape(B * F), tables.reshape(F * V // 8, 8 * D))
    return (
        op.transpose(0, 1, 3, 2, 4)
        .reshape(F, D, B)
        .transpose(2, 0, 1)
    )


# speedup vs baseline: 1.0098x; 1.0013x over previous
import jax
import jax.numpy as jnp
from jax import lax
from jax.experimental import pallas as pl
from jax.experimental.pallas import tpu as pltpu
from jax.experimental.pallas import tpu_sc as plsc

B = 16384
F = 26
V = 100000
D = 16

NW = 32
BPW = B // NW
CHUNK = 128
KPW = BPW // CHUNK
NBLK = F * KPW
NB = 4
DEPTH = 3


def _body(x_hbm, tab_hbm, op_hbm, xv, fidx, ridx, gbuf, tbuf, gsem, osem):
    cid = lax.axis_index("c")
    sid = lax.axis_index("s")
    wid = sid * 2 + cid
    b0 = wid * BPW

    pltpu.sync_copy(x_hbm.at[pl.ds(b0 * F, BPW * F)], xv)

    iota = lax.iota(jnp.int32, 16)
    zeros = jnp.zeros((16,), jnp.int32)
    iota_f = iota * F

    def build_and_fire(blk):
        f = lax.shift_right_logical(blk, 2)
        k = lax.bitwise_and(blk, 3)
        slot = lax.bitwise_and(blk, NB - 1)
        fv = f * V
        for q in range(8):
            addrs = iota_f + ((k * CHUNK + q * 16) * F + f)
            g = plsc.load_gather(xv, [addrs]) + fv
            fidx[slot, pl.ds(q * 16, 16)] = g
            ridx[slot, pl.ds(q * 16, 16)] = lax.shift_right_logical(g, 3)
        pltpu.async_copy(tab_hbm.at[ridx.at[slot]], gbuf.at[slot], gsem)

    def drain_and_emit(blk):
        f = lax.shift_right_logical(blk, 2)
        k = lax.bitwise_and(blk, 3)
        slot = lax.bitwise_and(blk, NB - 1)
        bt = wid * KPW + k
        pltpu.make_async_copy(
            tab_hbm.at[pl.ds(0, CHUNK)], gbuf.at[slot], gsem
        ).wait()
        @pl.when(blk >= NB)
        def _():
            pltpu.make_async_copy(
                op_hbm.at[0, 0, 0], tbuf.at[slot, pl.ds(0, 8)], osem
            ).wait()
            pltpu.make_async_copy(
                op_hbm.at[0, 0, 0], tbuf.at[slot, pl.ds(8, 8)], osem
            ).wait()

        def tr(q, carry):
            off = lax.bitwise_and(fidx[slot, pl.ds(q * 16, 16)], 7) * D
            rows = iota + q * 16
            for d in range(D):
                tbuf[slot, d, pl.ds(q * 16, 16)] = plsc.load_gather(
                    gbuf.at[slot], [rows, off + d]
                )
            return carry

        lax.fori_loop(0, 8, tr, 0)
        pltpu.async_copy(tbuf.at[slot, pl.ds(0, 8)], op_hbm.at[f, 0, bt], osem)
        pltpu.async_copy(tbuf.at[slot, pl.ds(8, 8)], op_hbm.at[f, 1, bt], osem)

    for blk in range(DEPTH):
        build_and_fire(jnp.int32(blk))

    def loop_body(blk, carry):
        build_and_fire(blk)
        drain_and_emit(blk - DEPTH)
        return carry

    lax.fori_loop(DEPTH, NBLK, loop_body, 0)
    for r in range(DEPTH):
        drain_and_emit(jnp.int32(NBLK - DEPTH + r))
    for r in range(NB):
        pltpu.make_async_copy(
            op_hbm.at[0, 0, 0], tbuf.at[r, pl.ds(0, 8)], osem
        ).wait()
        pltpu.make_async_copy(
            op_hbm.at[0, 0, 0], tbuf.at[r, pl.ds(8, 8)], osem
        ).wait()


def kernel(X, tables):
    mesh = plsc.VectorSubcoreMesh(core_axis_name="c", subcore_axis_name="s")
    op = pl.kernel(
        _body,
        mesh=mesh,
        out_type=jax.ShapeDtypeStruct((F, 2, B // CHUNK, 8, CHUNK), jnp.float32),
        scratch_types=[
            pltpu.VMEM((BPW * F,), jnp.int32),
            pltpu.VMEM((NB, CHUNK), jnp.int32),
            pltpu.VMEM((NB, CHUNK), jnp.int32),
            pltpu.VMEM((NB, CHUNK, CHUNK), jnp.float32),
            pltpu.VMEM((NB, D, CHUNK), jnp.float32),
            pltpu.SemaphoreType.DMA,
            pltpu.SemaphoreType.DMA,
        ],
        compiler_params=pltpu.CompilerParams(
            use_tc_tiling_on_sc=True, needs_layout_passes=False
        ),
    )(X.astype(jnp.int32).reshape(B * F), tables.reshape(F * V // 8, 8 * D))
    return (
        op.transpose(0, 1, 3, 2, 4)
        .reshape(F, D, B)
        .transpose(2, 0, 1)
    )
